# baseline (speedup 1.0000x reference)
import functools
import jax
import jax.numpy as jnp
from jax import lax
from jax.experimental import pallas as pl
from jax.experimental.pallas import tpu as pltpu
from jax.experimental.pallas import tpu_sc as plsc

NZ = 34
E = 7
NY = 256
NX = 512
NW = 32
ROWS_W = NY // NW
WSZ = E * NZ * NZ
GROUPS = NX // 16


def _sc_kernel(xq_hbm, xs_hbm, eta_hbm, wq_hbm, bq_hbm, ws_hbm, bs_hbm,
               out_hbm, wq_v, bq_v, ws_v, bs_v, xq_v, xs_v, eta_v,
               oq_v, os_v, sem):
    wid = lax.axis_index("s") * 2 + lax.axis_index("c")
    r0 = wid * ROWS_W
    pltpu.sync_copy(wq_hbm, wq_v)
    pltpu.sync_copy(bq_hbm, bq_v)
    pltpu.sync_copy(ws_hbm, ws_v)
    pltpu.sync_copy(bs_hbm, bs_v)

    def row_body(r, carry):
        row = r0 + r
        handles = []
        for w in range(NZ):
            handles.append(pltpu.async_copy(xq_hbm.at[w, row, :], xq_v.at[pl.ds(w * NX, NX)], sem))
            handles.append(pltpu.async_copy(xs_hbm.at[w, row, :], xs_v.at[pl.ds(w * NX, NX)], sem))
        handles.append(pltpu.async_copy(eta_hbm.at[row, :], eta_v, sem))
        for h in handles:
            h.wait()

        def group_body(g, carry2):
            col = pl.multiple_of(g * 16, 16)
            eta_vec = eta_v[pl.ds(col, 16)]
            wbase = eta_vec * (NZ * NZ)
            bbase = eta_vec * NZ
            for var in range(2):
                x_v = xq_v if var == 0 else xs_v
                w_v = wq_v if var == 0 else ws_v
                b_v = bq_v if var == 0 else bs_v
                o_v = oq_v if var == 0 else os_v
                xg = [x_v[pl.ds(w * NX + col, 16)] for w in range(NZ)]

                def z_body(z, carry3):
                    acc = plsc.load_gather(b_v, [bbase + z])
                    widx = wbase + z * NZ
                    for w in range(NZ):
                        acc = acc + plsc.load_gather(w_v, [widx + w]) * xg[w]
                    o_v[pl.ds(pl.multiple_of(z * NX + col, 16), 16)] = acc
                    return carry3

                lax.fori_loop(0, NZ, z_body, 0)
            return carry2

        lax.fori_loop(0, GROUPS, group_body, 0)
        ohandles = []
        for z in range(NZ):
            ohandles.append(pltpu.async_copy(oq_v.at[pl.ds(z * NX, NX)], out_hbm.at[0, z, row, :], sem))
            ohandles.append(pltpu.async_copy(os_v.at[pl.ds(z * NX, NX)], out_hbm.at[1, z, row, :], sem))
        for h in ohandles:
            h.wait()
        return carry

    lax.fori_loop(0, ROWS_W, row_body, 0)


def kernel(x_QT, x_SLI, eta, W_QT, b_QT, W_SLI, b_SLI):
    mesh = plsc.VectorSubcoreMesh(core_axis_name="c", subcore_axis_name="s")
    run = functools.partial(
        pl.kernel,
        out_type=jax.ShapeDtypeStruct((2, NZ, NY, NX), jnp.float32),
        mesh=mesh,
        compiler_params=pltpu.CompilerParams(needs_layout_passes=False),
        scratch_types=[
            pltpu.VMEM((WSZ,), jnp.float32),
            pltpu.VMEM((E * NZ,), jnp.float32),
            pltpu.VMEM((WSZ,), jnp.float32),
            pltpu.VMEM((E * NZ,), jnp.float32),
            pltpu.VMEM((NZ * NX,), jnp.float32),
            pltpu.VMEM((NZ * NX,), jnp.float32),
            pltpu.VMEM((NX,), jnp.int32),
            pltpu.VMEM((NZ * NX,), jnp.float32),
            pltpu.VMEM((NZ * NX,), jnp.float32),
            pltpu.SemaphoreType.DMA,
        ],
    )(_sc_kernel)
    return run(x_QT, x_SLI, eta,
               W_QT.reshape(WSZ), b_QT.reshape(E * NZ),
               W_SLI.reshape(WSZ), b_SLI.reshape(E * NZ))


# speedup vs baseline: 13.4128x; 13.4128x over previous
import jax
import jax.numpy as jnp
from jax.experimental import pallas as pl
from jax.experimental.pallas import tpu as pltpu

NZ = 34
E = 7
S = 40
KX = E * S
R = 16


def _moe_kernel(eta_ref, xq_ref, xs_ref, wq_ref, ws_ref, out_ref):
    _, ny, nx = xq_ref.shape
    n = ny * nx
    eta = eta_ref[0]
    xq = xq_ref[...].astype(jnp.bfloat16).reshape(NZ, n)
    xs = xs_ref[...].astype(jnp.bfloat16).reshape(NZ, n)
    pad = jnp.zeros((S - NZ - 1, n), jnp.bfloat16)
    one = jnp.ones((1, n), jnp.bfloat16)
    xaugq = jnp.concatenate([xq, one, pad], axis=0)
    xaugs = jnp.concatenate([xs, one, pad], axis=0)
    zed = jnp.zeros((S, n), jnp.bfloat16)
    xkq = jnp.concatenate([jnp.where(eta == e, xaugq, zed) for e in range(E)], axis=0)
    xks = jnp.concatenate([jnp.where(eta == e, xaugs, zed) for e in range(E)], axis=0)
    oq = jnp.dot(wq_ref[...], xkq, preferred_element_type=jnp.float32)
    osli = jnp.dot(ws_ref[...], xks, preferred_element_type=jnp.float32)
    out_ref[0] = oq.reshape(NZ, ny, nx)
    out_ref[1] = osli.reshape(NZ, ny, nx)


def _stack_weights(W, b):
    wc = jnp.zeros((NZ, E, S), W.dtype)
    wc = wc.at[:, :, :NZ].set(jnp.transpose(W, (1, 0, 2)))
    wc = wc.at[:, :, NZ].set(b.T)
    return wc.reshape(NZ, KX).astype(jnp.bfloat16)


def kernel(x_QT, x_SLI, eta, W_QT, b_QT, W_SLI, b_SLI):
    NY, NX = eta.shape
    G = NY // R
    wq = _stack_weights(W_QT, b_QT)
    ws = _stack_weights(W_SLI, b_SLI)
    eta3 = eta.reshape(G, 1, R * NX)
    return pl.pallas_call(
        _moe_kernel,
        grid=(G,),
        in_specs=[
            pl.BlockSpec((1, 1, R * NX), lambda i: (i, 0, 0)),
            pl.BlockSpec((NZ, R, NX), lambda i: (0, i, 0)),
            pl.BlockSpec((NZ, R, NX), lambda i: (0, i, 0)),
            pl.BlockSpec((NZ, KX), lambda i: (0, 0)),
            pl.BlockSpec((NZ, KX), lambda i: (0, 0)),
        ],
        out_specs=pl.BlockSpec((2, NZ, R, NX), lambda i: (0, 0, i, 0)),
        out_shape=jax.ShapeDtypeStruct((2, NZ, NY, NX), jnp.float32),
        compiler_params=pltpu.CompilerParams(
            dimension_semantics=("parallel",)),
    )(eta3, x_QT, x_SLI, wq, ws)


# speedup vs baseline: 13.9100x; 1.0371x over previous
import jax
import jax.numpy as jnp
from jax.experimental import pallas as pl
from jax.experimental.pallas import tpu as pltpu

NZ = 34
E = 7
S = 40
KX = E * S
R = 16


def _moe_kernel(eta_ref, xq_ref, xs_ref, wq_ref, ws_ref, out_ref):
    _, ny, nx = xq_ref.shape
    n = ny * nx
    eta = eta_ref[...].reshape(1, n)
    xq = xq_ref[...].astype(jnp.bfloat16).reshape(NZ, n)
    xs = xs_ref[...].astype(jnp.bfloat16).reshape(NZ, n)
    pad = jnp.zeros((S - NZ - 1, n), jnp.bfloat16)
    one = jnp.ones((1, n), jnp.bfloat16)
    xaugq = jnp.concatenate([xq, one, pad], axis=0)
    xaugs = jnp.concatenate([xs, one, pad], axis=0)
    zed = jnp.zeros((S, n), jnp.bfloat16)
    xkq = jnp.concatenate([jnp.where(eta == e, xaugq, zed) for e in range(E)], axis=0)
    xks = jnp.concatenate([jnp.where(eta == e, xaugs, zed) for e in range(E)], axis=0)
    oq = jnp.dot(wq_ref[...], xkq, preferred_element_type=jnp.float32)
    osli = jnp.dot(ws_ref[...], xks, preferred_element_type=jnp.float32)
    out_ref[0] = oq.reshape(NZ, ny, nx)
    out_ref[1] = osli.reshape(NZ, ny, nx)


def _stack_weights(W, b):
    wc = jnp.zeros((NZ, E, S), W.dtype)
    wc = wc.at[:, :, :NZ].set(jnp.transpose(W, (1, 0, 2)))
    wc = wc.at[:, :, NZ].set(b.T)
    return wc.reshape(NZ, KX).astype(jnp.bfloat16)


def kernel(x_QT, x_SLI, eta, W_QT, b_QT, W_SLI, b_SLI):
    NY, NX = eta.shape
    G = NY // R
    wq = _stack_weights(W_QT, b_QT)
    ws = _stack_weights(W_SLI, b_SLI)
    return pl.pallas_call(
        _moe_kernel,
        grid=(G,),
        in_specs=[
            pl.BlockSpec((R, NX), lambda i: (i, 0)),
            pl.BlockSpec((NZ, R, NX), lambda i: (0, i, 0)),
            pl.BlockSpec((NZ, R, NX), lambda i: (0, i, 0)),
            pl.BlockSpec((NZ, KX), lambda i: (0, 0)),
            pl.BlockSpec((NZ, KX), lambda i: (0, 0)),
        ],
        out_specs=pl.BlockSpec((2, NZ, R, NX), lambda i: (0, 0, i, 0)),
        out_shape=jax.ShapeDtypeStruct((2, NZ, NY, NX), jnp.float32),
        compiler_params=pltpu.CompilerParams(
            dimension_semantics=("parallel",)),
    )(eta, x_QT, x_SLI, wq, ws)


# speedup vs baseline: 15.3542x; 1.1038x over previous
import jax
import jax.numpy as jnp
from jax.experimental import pallas as pl
from jax.experimental.pallas import tpu as pltpu

NZ = 34
E = 7
S = 40
KX = E * S
R = 16


def _moe_kernel(eta_ref, xq_ref, xs_ref, wq_ref, ws_ref, out_ref):
    _, ny, nx = xq_ref.shape
    n = ny * nx
    eta = eta_ref[...].reshape(1, n)
    xq = xq_ref[...].astype(jnp.bfloat16).reshape(NZ, n)
    xs = xs_ref[...].astype(jnp.bfloat16).reshape(NZ, n)
    pad = jnp.zeros((S - NZ - 1, n), jnp.bfloat16)
    one = jnp.ones((1, n), jnp.bfloat16)
    xaugq = jnp.concatenate([xq, one, pad], axis=0)
    xaugs = jnp.concatenate([xs, one, pad], axis=0)
    zed = jnp.zeros((S, n), jnp.bfloat16)
    xkq = jnp.concatenate([jnp.where(eta == e, xaugq, zed) for e in range(E)], axis=0)
    xks = jnp.concatenate([jnp.where(eta == e, xaugs, zed) for e in range(E)], axis=0)
    oq = jnp.dot(wq_ref[...], xkq, preferred_element_type=jnp.float32)
    osli = jnp.dot(ws_ref[...], xks, preferred_element_type=jnp.float32)
    out_ref[0] = oq.reshape(NZ, ny, nx)
    out_ref[1] = osli.reshape(NZ, ny, nx)


def _stack_weights(W, b):
    pad = jnp.zeros((E, S - NZ - 1, NZ), W.dtype)
    wt = jnp.concatenate([W, b[:, None, :], pad], axis=1)
    return wt.reshape(KX, NZ).T.astype(jnp.bfloat16)


def kernel(x_QT, x_SLI, eta, W_QT, b_QT, W_SLI, b_SLI):
    NY, NX = eta.shape
    G = NY // R
    wq = _stack_weights(W_QT, b_QT)
    ws = _stack_weights(W_SLI, b_SLI)
    return pl.pallas_call(
        _moe_kernel,
        grid=(G,),
        in_specs=[
            pl.BlockSpec((R, NX), lambda i: (i, 0)),
            pl.BlockSpec((NZ, R, NX), lambda i: (0, i, 0)),
            pl.BlockSpec((NZ, R, NX), lambda i: (0, i, 0)),
            pl.BlockSpec((NZ, KX), lambda i: (0, 0)),
            pl.BlockSpec((NZ, KX), lambda i: (0, 0)),
        ],
        out_specs=pl.BlockSpec((2, NZ, R, NX), lambda i: (0, 0, i, 0)),
        out_shape=jax.ShapeDtypeStruct((2, NZ, NY, NX), jnp.float32),
        compiler_params=pltpu.CompilerParams(
            dimension_semantics=("parallel",)),
    )(eta, x_QT, x_SLI, wq, ws)


# speedup vs baseline: 15.4076x; 1.0035x over previous
import jax
import jax.numpy as jnp
from jax.experimental import pallas as pl
from jax.experimental.pallas import tpu as pltpu

NZ = 34
E = 7
S = 40
KX = E * S
R = 16


def _moe_kernel(eta_ref, xq_ref, xs_ref, wq_ref, ws_ref, out_ref):
    _, ny, nx = xq_ref.shape
    n = ny * nx
    eta = eta_ref[...].reshape(1, n)
    xq = xq_ref[...].astype(jnp.bfloat16).reshape(NZ, n)
    xs = xs_ref[...].astype(jnp.bfloat16).reshape(NZ, n)
    pad = jnp.zeros((S - NZ - 1, n), jnp.bfloat16)
    one = jnp.ones((1, n), jnp.bfloat16)
    xaugq = jnp.concatenate([xq, one, pad], axis=0)
    xaugs = jnp.concatenate([xs, one, pad], axis=0)
    zed = jnp.zeros((S, n), jnp.bfloat16)
    xkq = jnp.concatenate([jnp.where(eta == e, xaugq, zed) for e in range(E)], axis=0)
    xks = jnp.concatenate([jnp.where(eta == e, xaugs, zed) for e in range(E)], axis=0)
    oq = jnp.dot(wq_ref[...], xkq, preferred_element_type=jnp.float32)
    osli = jnp.dot(ws_ref[...], xks, preferred_element_type=jnp.float32)
    out_ref[0] = oq.reshape(NZ, ny, nx)
    out_ref[1] = osli.reshape(NZ, ny, nx)


def _stack_weights(W, b):
    pad = jnp.zeros((E, S - NZ - 1, NZ), W.dtype)
    wt = jnp.concatenate([jnp.swapaxes(W, 1, 2), b[:, None, :], pad], axis=1)
    return wt.reshape(KX, NZ).T.astype(jnp.bfloat16)


def kernel(x_QT, x_SLI, eta, W_QT, b_QT, W_SLI, b_SLI):
    NY, NX = eta.shape
    G = NY // R
    wq = _stack_weights(W_QT, b_QT)
    ws = _stack_weights(W_SLI, b_SLI)
    return pl.pallas_call(
        _moe_kernel,
        grid=(G,),
        in_specs=[
            pl.BlockSpec((R, NX), lambda i: (i, 0)),
            pl.BlockSpec((NZ, R, NX), lambda i: (0, i, 0)),
            pl.BlockSpec((NZ, R, NX), lambda i: (0, i, 0)),
            pl.BlockSpec((NZ, KX), lambda i: (0, 0)),
            pl.BlockSpec((NZ, KX), lambda i: (0, 0)),
        ],
        out_specs=pl.BlockSpec((2, NZ, R, NX), lambda i: (0, 0, i, 0)),
        out_shape=jax.ShapeDtypeStruct((2, NZ, NY, NX), jnp.float32),
        compiler_params=pltpu.CompilerParams(
            dimension_semantics=("parallel",)),
    )(eta, x_QT, x_SLI, wq, ws)
